# baseline (speedup 1.0000x reference)
import dataclasses
import functools

import jax
import jax.numpy as jnp
from jax import lax
from jax.experimental import pallas as pl
from jax.experimental.pallas import tpu as pltpu
from jax.experimental.pallas import tpu_sc as plsc

B = 64
U = 16
D = 256
K = 32
C = 32
HW = 1024
F = C * HW
DEPTH = 4
UK = U * K
UD = U * D
CBLK = 8
NKB = C // CBLK


def _hash_route_kernel(cast_hw, *refs):
    if cast_hw:
        (x_ref, hw_ref, b2_ref, b2t_ref,
         h_ref, p2_ref, mags_ref, hwbf_ref) = refs
    else:
        (x_ref, hw_ref, b2_ref, b2t_ref,
         h_ref, p2_ref, mags_ref) = refs
    kg = pl.program_id(0)

    @pl.when(kg == 0)
    def _init():
        h_ref[...] = jnp.zeros_like(h_ref)

    if cast_hw:
        hwc = hw_ref[...].astype(jnp.bfloat16)
        hwbf_ref[...] = hwc
    else:
        hwc = hw_ref[...]
    acc0 = jnp.zeros((B, D), jnp.float32)
    acc1 = jnp.zeros((B, D), jnp.float32)
    for j in range(CBLK):
        xc = x_ref[:, j, :].astype(jnp.bfloat16)
        p = jnp.dot(xc, hwc[j * HW:(j + 1) * HW, :],
                    preferred_element_type=jnp.float32)
        if j % 2 == 0:
            acc0 = acc0 + p
        else:
            acc1 = acc1 + p
    h_ref[...] += acc0 + acc1

    @pl.when(kg == NKB - 1)
    def _fin():
        h = h_ref[...]
        coef = jnp.dot(h.astype(jnp.bfloat16), b2_ref[...],
                       preferred_element_type=jnp.float32)
        cbf = coef.astype(jnp.bfloat16)
        m2 = []
        for u in range(U):
            pu = jnp.dot(cbf[:, u * K:(u + 1) * K],
                         b2t_ref[u * K:(u + 1) * K, :],
                         preferred_element_type=jnp.float32)
            p2_ref[:, u * D:(u + 1) * D] = pu
            m2.append(jnp.sum(pu * pu, axis=1))
        mags_ref[:, 0:U] = jnp.stack(m2, axis=1)


def _hash_route_call(x3, hw, B2bf, B2Tbf, cast_hw=False):
    full = lambda k: (0, 0)
    out_specs = [
        pl.BlockSpec((B, D), full),
        pl.BlockSpec((B, UD), full),
        pl.BlockSpec((B, 128), full),
    ]
    out_shape = [
        jax.ShapeDtypeStruct((B, D), jnp.float32),
        jax.ShapeDtypeStruct((B, UD), jnp.float32),
        jax.ShapeDtypeStruct((B, 128), jnp.float32),
    ]
    if cast_hw:
        out_specs.append(pl.BlockSpec((CBLK * HW, D), lambda k: (k, 0)))
        out_shape.append(jax.ShapeDtypeStruct((F, D), jnp.bfloat16))
    return pl.pallas_call(
        functools.partial(_hash_route_kernel, cast_hw),
        grid=(NKB,),
        in_specs=[
            pl.BlockSpec((B, CBLK, HW), lambda k: (0, k, 0)),
            pl.BlockSpec((CBLK * HW, D), lambda k: (k, 0)),
            pl.BlockSpec((D, UK), full),
            pl.BlockSpec((UK, D), full),
        ],
        out_specs=out_specs,
        out_shape=out_shape,
    )(x3, hw, B2bf, B2Tbf)


NW = 32
EPW = B // NW


def _sc_route_call(d_const, depth16, magsp_f, h_f, p2_f, avail_f, outp_f,
                   act_f):
    mesh = plsc.VectorSubcoreMesh(core_axis_name="core",
                                  subcore_axis_name="subcore")
    cp = pltpu.CompilerParams()
    if "needs_layout_passes" in pltpu.CompilerParams.__dataclass_fields__:
        cp = dataclasses.replace(cp, needs_layout_passes=False)

    @functools.partial(
        pl.kernel, mesh=mesh, compiler_params=cp,
        out_type=[
            jax.ShapeDtypeStruct((B * U,), jnp.int32),
            jax.ShapeDtypeStruct((B * U,), jnp.int32),
            jax.ShapeDtypeStruct((B * U,), jnp.float32),
            jax.ShapeDtypeStruct((B * D,), jnp.float32),
            jax.ShapeDtypeStruct((B * U,), jnp.float32),
        ],
        scratch_types=[
            pltpu.VMEM((U,), jnp.int32),
            pltpu.VMEM((U,), jnp.float32),
            pltpu.VMEM((U,), jnp.float32),
            pltpu.VMEM((U,), jnp.float32),
            pltpu.VMEM((D,), jnp.float32),
            pltpu.VMEM((D,), jnp.float32),
            pltpu.VMEM((UD,), jnp.float32),
            pltpu.VMEM((U,), jnp.int32),
            pltpu.VMEM((U,), jnp.int32),
            pltpu.VMEM((U,), jnp.float32),
            pltpu.VMEM((D,), jnp.float32),
            pltpu.VMEM((U,), jnp.float32),
        ],
    )
    def sc_route(depth_hbm, mags_hbm, h_hbm, p2_hbm, avail_hbm, outp_hbm,
                 act_hbm, route_o, idx_o, avail_o, outp_o, act_o,
                 d_v, m_v, a_v, ac_v, h_v, o_v, p2_v,
                 rt_v, ix_v, an_v, on_v, acn_v):
        wid = lax.axis_index("subcore") * 2 + lax.axis_index("core")
        pltpu.sync_copy(depth_hbm, d_v)
        son = d_const < d_v[...]
        sonf = jnp.where(son, 1.0, 0.0)
        iota = lax.broadcasted_iota(jnp.int32, (U,), 0)
        for e in range(EPW):
            b = wid * EPW + e
            pltpu.sync_copy(mags_hbm.at[pl.ds(b * 128, U)], m_v)
            pltpu.sync_copy(avail_hbm.at[pl.ds(b * U, U)], a_v)
            pltpu.sync_copy(act_hbm.at[pl.ds(b * U, U)], ac_v)
            pltpu.sync_copy(h_hbm.at[pl.ds(b * D, D)], h_v)
            pltpu.sync_copy(outp_hbm.at[pl.ds(b * D, D)], o_v)
            pltpu.sync_copy(p2_hbm.at[pl.ds(b * UD, UD)], p2_v)
            mags = m_v[...]
            avail = a_v[...]
            masked = jnp.where(avail > 0.5, mags, -jnp.inf)
            mx = jnp.max(masked)
            eq = masked == mx
            ffs = plsc.all_reduce_ffs(eq)
            idxv = jnp.zeros((U,), jnp.int32) + ffs
            onehot = iota == idxv
            act = ac_v[...]
            actson = act * sonf
            rn2 = jnp.zeros((U,), jnp.float32)
            for j in range(D // U):
                hj = h_v[pl.ds(U * j, U)]
                pj = plsc.load_gather(p2_v, [idxv * D + U * j + iota])
                r = hj - pj
                rn2 = rn2 + r * r
                on_v[pl.ds(U * j, U)] = o_v[pl.ds(U * j, U)] + r * actson
            rtot = jnp.sum(rn2)
            live = rtot > 1e-10
            acn_v[...] = jnp.where(son, act * jnp.where(live, 1.0, 0.0), act)
            an_v[...] = jnp.where(son & onehot, 0.0, avail)
            rt_v[...] = jnp.where(son, idxv, 0)
            ix_v[...] = idxv
            pltpu.sync_copy(rt_v, route_o.at[pl.ds(b * U, U)])
            pltpu.sync_copy(ix_v, idx_o.at[pl.ds(b * U, U)])
            pltpu.sync_copy(an_v, avail_o.at[pl.ds(b * U, U)])
            pltpu.sync_copy(on_v, outp_o.at[pl.ds(b * D, D)])
            pltpu.sync_copy(acn_v, act_o.at[pl.ds(b * U, U)])

    return sc_route(depth16, magsp_f, h_f, p2_f, avail_f, outp_f, act_f)


GB = 8


def _conv_kernel(idx_ref, x_ref, w_ref, o_ref):
    g = pl.program_id(0)
    wpos = lax.broadcasted_iota(jnp.int32, (C, HW), 1) % 32
    mnr = (wpos != 31).astype(jnp.bfloat16)
    mnl = (wpos != 0).astype(jnp.bfloat16)
    for e in range(GB):
        ib = idx_ref[(g * GB + e) * U]
        xb = x_ref[e].astype(jnp.bfloat16)
        acc = jnp.zeros((C, HW), jnp.float32)
        acc1 = jnp.zeros((C, HW), jnp.float32)
        for t in range(9):
            ky, kx = t // 3, t % 3
            dy, dx = ky - 1, kx - 1
            sh = 32 * dy + dx
            if sh > 0:
                s = jnp.concatenate(
                    [xb[:, sh:], jnp.zeros((C, sh), jnp.bfloat16)], axis=1)
            elif sh < 0:
                s = jnp.concatenate(
                    [jnp.zeros((C, -sh), jnp.bfloat16), xb[:, :sh]], axis=1)
            else:
                s = xb
            if dx == 1:
                s = s * mnr
            elif dx == -1:
                s = s * mnl
            p = jnp.dot(w_ref[ib, t], s,
                        preferred_element_type=jnp.float32)
            if t % 2 == 0:
                acc = acc + p
            else:
                acc1 = acc1 + p
        o_ref[e] = jnp.maximum(acc + acc1, 0.0)


def _conv_call(idx, x3, CWr_bf):
    return pl.pallas_call(
        _conv_kernel,
        grid_spec=pltpu.PrefetchScalarGridSpec(
            num_scalar_prefetch=1,
            grid=(B // GB,),
            in_specs=[
                pl.BlockSpec((GB, C, HW), lambda g, idx_ref: (g, 0, 0)),
                pl.BlockSpec((U, 9, C, C),
                             lambda g, idx_ref: (0, 0, 0, 0)),
            ],
            out_specs=pl.BlockSpec((GB, C, HW), lambda g, idx_ref: (g, 0, 0)),
        ),
        out_shape=jax.ShapeDtypeStruct((B, C, HW), jnp.float32),
    )(idx, x3, CWr_bf)


def kernel(x, hash_W, bases, conv_W, depth):
    depth16 = jnp.full((U,), depth, jnp.int32)
    x3 = x.reshape(B, C, HW)
    B2bf = bases.transpose(1, 0, 2).reshape(D, UK).astype(jnp.bfloat16)
    B2Tbf = bases.transpose(0, 2, 1).reshape(UK, D).astype(jnp.bfloat16)
    CWr_bf = conv_W.transpose(0, 3, 4, 1, 2).reshape(U, 9, C, C).astype(
        jnp.bfloat16)

    outp_f = jnp.zeros((B * D,), jnp.float32)
    avail_f = jnp.ones((B * U,), jnp.float32)
    act_f = jnp.ones((B * U,), jnp.float32)
    routes = []
    xcur = x3
    hw_bf = None
    for d in range(DEPTH):
        if d == 0:
            h, p2, magsp, hw_bf = _hash_route_call(
                xcur, hash_W, B2bf, B2Tbf, cast_hw=True)
        else:
            h, p2, magsp = _hash_route_call(xcur, hw_bf, B2bf, B2Tbf)
        route_f, idx_f, avail_f, outp_f, act_f = _sc_route_call(
            d, depth16, magsp.reshape(B * 128), h.reshape(B * D),
            p2.reshape(B * UD), avail_f, outp_f, act_f)
        routes.append(route_f.reshape(B, U)[:, 0])
        if d < DEPTH - 1:
            xcur = _conv_call(idx_f, xcur, CWr_bf)
    return outp_f.reshape(B, D), jnp.stack(routes, axis=1)


# speedup vs baseline: 1.1487x; 1.1487x over previous
import dataclasses
import functools

import jax
import jax.numpy as jnp
from jax import lax
from jax.experimental import pallas as pl
from jax.experimental.pallas import tpu as pltpu
from jax.experimental.pallas import tpu_sc as plsc

B = 64
U = 16
D = 256
K = 32
C = 32
HW = 1024
F = C * HW
DEPTH = 4
UK = U * K
UD = U * D
CBLK = 8
NKB = C // CBLK


def _hash_route_kernel(d_const, cast_hw, *refs):
    if cast_hw:
        (depth_ref, x_ref, hw_ref, b2_ref, b2t_ref, avail_ref,
         h_ref, p2_ref, idxp_ref, idx2_ref, route_ref, avail_o_ref,
         hwbf_ref) = refs
    else:
        (depth_ref, x_ref, hw_ref, b2_ref, b2t_ref, avail_ref,
         h_ref, p2_ref, idxp_ref, idx2_ref, route_ref, avail_o_ref) = refs
    kg = pl.program_id(0)

    @pl.when(kg == 0)
    def _init():
        h_ref[...] = jnp.zeros_like(h_ref)

    if cast_hw:
        hwc = hw_ref[...].astype(jnp.bfloat16)
        hwbf_ref[...] = hwc
    else:
        hwc = hw_ref[...]
    acc0 = jnp.zeros((B, D), jnp.float32)
    acc1 = jnp.zeros((B, D), jnp.float32)
    for j in range(CBLK):
        xc = x_ref[:, j, :].astype(jnp.bfloat16)
        p = jnp.dot(xc, hwc[j * HW:(j + 1) * HW, :],
                    preferred_element_type=jnp.float32)
        if j % 2 == 0:
            acc0 = acc0 + p
        else:
            acc1 = acc1 + p
    h_ref[...] += acc0 + acc1

    @pl.when(kg == NKB - 1)
    def _fin():
        h = h_ref[...]
        coef = jnp.dot(h.astype(jnp.bfloat16), b2_ref[...],
                       preferred_element_type=jnp.float32)
        cbf = coef.astype(jnp.bfloat16)
        m2 = []
        for u in range(U):
            pu = jnp.dot(cbf[:, u * K:(u + 1) * K],
                         b2t_ref[u * K:(u + 1) * K, :],
                         preferred_element_type=jnp.float32)
            p2_ref[:, u * D:(u + 1) * D] = pu
            m2.append(jnp.sum(pu * pu, axis=1))
        mags2 = jnp.stack(m2, axis=1)
        step_on = d_const < depth_ref[0]
        avail = avail_ref[...]
        masked = jnp.where(avail > 0.5, mags2, -jnp.inf)
        m = jnp.max(masked, axis=1, keepdims=True)
        iota = lax.broadcasted_iota(jnp.int32, (B, U), 1)
        idx = jnp.min(jnp.where(masked == m, iota, U + 1), axis=1)
        onehot = iota == idx[:, None]
        avail_o_ref[...] = jnp.where(
            jnp.logical_and(step_on, onehot), 0.0, avail)
        idxp_ref[:, 0:U] = jnp.zeros((B, U), jnp.int32) + idx[:, None]
        idx2_ref[...] = idx[:, None]
        route_ref[...] = jnp.where(step_on, idx, 0)[:, None]


def _hash_route_call(d, depth_arr, x3, hw, B2bf, B2Tbf, avail, cast_hw=False):
    full = lambda k: (0, 0)
    out_specs = [
        pl.BlockSpec((B, D), full),
        pl.BlockSpec((B, UD), full),
        pl.BlockSpec((B, 128), full),
        pl.BlockSpec((B, 1), full),
        pl.BlockSpec((B, 1), full),
        pl.BlockSpec((B, U), full),
    ]
    out_shape = [
        jax.ShapeDtypeStruct((B, D), jnp.float32),
        jax.ShapeDtypeStruct((B, UD), jnp.float32),
        jax.ShapeDtypeStruct((B, 128), jnp.int32),
        jax.ShapeDtypeStruct((B, 1), jnp.int32),
        jax.ShapeDtypeStruct((B, 1), jnp.int32),
        jax.ShapeDtypeStruct((B, U), jnp.float32),
    ]
    if cast_hw:
        out_specs.append(pl.BlockSpec((CBLK * HW, D), lambda k: (k, 0)))
        out_shape.append(jax.ShapeDtypeStruct((F, D), jnp.bfloat16))
    return pl.pallas_call(
        functools.partial(_hash_route_kernel, d, cast_hw),
        grid=(NKB,),
        in_specs=[
            pl.BlockSpec(memory_space=pltpu.SMEM),
            pl.BlockSpec((B, CBLK, HW), lambda k: (0, k, 0)),
            pl.BlockSpec((CBLK * HW, D), lambda k: (k, 0)),
            pl.BlockSpec((D, UK), full),
            pl.BlockSpec((UK, D), full),
            pl.BlockSpec((B, U), full),
        ],
        out_specs=out_specs,
        out_shape=out_shape,
    )(depth_arr, x3, hw, B2bf, B2Tbf, avail)


NW = 32
EPW = B // NW


def _sc_accum_call(d_const, depth16, idxp_f, h_f, p2_f, outp_f, act_f):
    mesh = plsc.VectorSubcoreMesh(core_axis_name="core",
                                  subcore_axis_name="subcore")
    cp = pltpu.CompilerParams()
    if "needs_layout_passes" in pltpu.CompilerParams.__dataclass_fields__:
        cp = dataclasses.replace(cp, needs_layout_passes=False)

    @functools.partial(
        pl.kernel, mesh=mesh, compiler_params=cp,
        out_type=[
            jax.ShapeDtypeStruct((B * D,), jnp.float32),
            jax.ShapeDtypeStruct((B * U,), jnp.float32),
        ],
        scratch_types=[
            pltpu.VMEM((U,), jnp.int32),
            pltpu.VMEM((U,), jnp.int32),
            pltpu.VMEM((U,), jnp.float32),
            pltpu.VMEM((D,), jnp.float32),
            pltpu.VMEM((D,), jnp.float32),
            pltpu.VMEM((UD,), jnp.float32),
            pltpu.VMEM((D,), jnp.float32),
            pltpu.VMEM((U,), jnp.float32),
        ],
    )
    def sc_accum(depth_hbm, idxp_hbm, h_hbm, p2_hbm, outp_hbm,
                 act_hbm, outp_o, act_o,
                 d_v, ix_v, ac_v, h_v, o_v, p2_v, on_v, acn_v):
        wid = lax.axis_index("subcore") * 2 + lax.axis_index("core")
        pltpu.sync_copy(depth_hbm, d_v)
        son = d_const < d_v[...]
        sonf = jnp.where(son, 1.0, 0.0)
        iota = lax.broadcasted_iota(jnp.int32, (U,), 0)
        for e in range(EPW):
            b = wid * EPW + e
            pltpu.sync_copy(idxp_hbm.at[pl.ds(b * 128, U)], ix_v)
            pltpu.sync_copy(act_hbm.at[pl.ds(b * U, U)], ac_v)
            pltpu.sync_copy(h_hbm.at[pl.ds(b * D, D)], h_v)
            pltpu.sync_copy(outp_hbm.at[pl.ds(b * D, D)], o_v)
            pltpu.sync_copy(p2_hbm.at[pl.ds(b * UD, UD)], p2_v)
            idxv = ix_v[...]
            act = ac_v[...]
            actson = act * sonf
            rn2 = jnp.zeros((U,), jnp.float32)
            for j in range(D // U):
                hj = h_v[pl.ds(U * j, U)]
                pj = plsc.load_gather(p2_v, [idxv * D + U * j + iota])
                r = hj - pj
                rn2 = rn2 + r * r
                on_v[pl.ds(U * j, U)] = o_v[pl.ds(U * j, U)] + r * actson
            rtot = jnp.sum(rn2)
            live = rtot > 1e-10
            acn_v[...] = jnp.where(son, act * jnp.where(live, 1.0, 0.0), act)
            pltpu.sync_copy(on_v, outp_o.at[pl.ds(b * D, D)])
            pltpu.sync_copy(acn_v, act_o.at[pl.ds(b * U, U)])

    return sc_accum(depth16, idxp_f, h_f, p2_f, outp_f, act_f)


GB = 8


def _conv_kernel(idx_ref, x_ref, w_ref, o_ref):
    g = pl.program_id(0)
    wpos = lax.broadcasted_iota(jnp.int32, (C, HW), 1) % 32
    mnr = (wpos != 31).astype(jnp.bfloat16)
    mnl = (wpos != 0).astype(jnp.bfloat16)
    for e in range(GB):
        ib = idx_ref[g * GB + e]
        xb = x_ref[e].astype(jnp.bfloat16)
        acc = jnp.zeros((C, HW), jnp.float32)
        acc1 = jnp.zeros((C, HW), jnp.float32)
        for t in range(9):
            ky, kx = t // 3, t % 3
            dy, dx = ky - 1, kx - 1
            sh = 32 * dy + dx
            if sh > 0:
                s = jnp.concatenate(
                    [xb[:, sh:], jnp.zeros((C, sh), jnp.bfloat16)], axis=1)
            elif sh < 0:
                s = jnp.concatenate(
                    [jnp.zeros((C, -sh), jnp.bfloat16), xb[:, :sh]], axis=1)
            else:
                s = xb
            if dx == 1:
                s = s * mnr
            elif dx == -1:
                s = s * mnl
            p = jnp.dot(w_ref[ib, t], s,
                        preferred_element_type=jnp.float32)
            if t % 2 == 0:
                acc = acc + p
            else:
                acc1 = acc1 + p
        o_ref[e] = jnp.maximum(acc + acc1, 0.0)


def _conv_call(idx, x3, CWr_bf):
    return pl.pallas_call(
        _conv_kernel,
        grid_spec=pltpu.PrefetchScalarGridSpec(
            num_scalar_prefetch=1,
            grid=(B // GB,),
            in_specs=[
                pl.BlockSpec((GB, C, HW), lambda g, idx_ref: (g, 0, 0)),
                pl.BlockSpec((U, 9, C, C),
                             lambda g, idx_ref: (0, 0, 0, 0)),
            ],
            out_specs=pl.BlockSpec((GB, C, HW), lambda g, idx_ref: (g, 0, 0)),
        ),
        out_shape=jax.ShapeDtypeStruct((B, C, HW), jnp.float32),
    )(idx, x3, CWr_bf)


def kernel(x, hash_W, bases, conv_W, depth):
    depth16 = jnp.full((U,), depth, jnp.int32)
    depth_arr = jnp.asarray(depth, jnp.int32).reshape(1)
    x3 = x.reshape(B, C, HW)
    B2bf = bases.transpose(1, 0, 2).reshape(D, UK).astype(jnp.bfloat16)
    B2Tbf = bases.transpose(0, 2, 1).reshape(UK, D).astype(jnp.bfloat16)
    CWr_bf = conv_W.transpose(0, 3, 4, 1, 2).reshape(U, 9, C, C).astype(
        jnp.bfloat16)

    outp_f = jnp.zeros((B * D,), jnp.float32)
    avail = jnp.ones((B, U), jnp.float32)
    act_f = jnp.ones((B * U,), jnp.float32)
    routes = []
    xcur = x3
    hw_bf = None
    for d in range(DEPTH):
        if d == 0:
            h, p2, idxp, idx2, route_d, avail, hw_bf = _hash_route_call(
                d, depth_arr, xcur, hash_W, B2bf, B2Tbf, avail, cast_hw=True)
        else:
            h, p2, idxp, idx2, route_d, avail = _hash_route_call(
                d, depth_arr, xcur, hw_bf, B2bf, B2Tbf, avail)
        outp_f, act_f = _sc_accum_call(
            d, depth16, idxp.reshape(B * 128), h.reshape(B * D),
            p2.reshape(B * UD), outp_f, act_f)
        routes.append(route_d[:, 0])
        if d < DEPTH - 1:
            xcur = _conv_call(idx2[:, 0], xcur, CWr_bf)
    return outp_f.reshape(B, D), jnp.stack(routes, axis=1)


# speedup vs baseline: 1.1784x; 1.0258x over previous
import dataclasses
import functools

import jax
import jax.numpy as jnp
from jax import lax
from jax.experimental import pallas as pl
from jax.experimental.pallas import tpu as pltpu
from jax.experimental.pallas import tpu_sc as plsc

B = 64
U = 16
D = 256
K = 32
C = 32
HW = 1024
F = C * HW
DEPTH = 4
UK = U * K
UD = U * D
CBLK = 8
NKB = C // CBLK


def _hash_route_kernel(d_const, cast_hw, *refs):
    if cast_hw:
        (depth_ref, x_ref, hw_ref, b2_ref, b2t_ref, avail_ref,
         h_ref, p2_ref, idxp_ref, idx2_ref, route_ref, avail_o_ref,
         hwbf_ref) = refs
    else:
        (depth_ref, x_ref, hw_ref, b2_ref, b2t_ref, avail_ref,
         h_ref, p2_ref, idxp_ref, idx2_ref, route_ref, avail_o_ref) = refs
    kg = pl.program_id(0)

    @pl.when(kg == 0)
    def _init():
        h_ref[...] = jnp.zeros_like(h_ref)

    if cast_hw:
        hwc = hw_ref[...].astype(jnp.bfloat16)
        hwbf_ref[...] = hwc
    else:
        hwc = hw_ref[...]
    acc0 = jnp.zeros((B, D), jnp.float32)
    acc1 = jnp.zeros((B, D), jnp.float32)
    for j in range(CBLK):
        xc = x_ref[:, j, :].astype(jnp.bfloat16)
        p = jnp.dot(xc, hwc[j * HW:(j + 1) * HW, :],
                    preferred_element_type=jnp.float32)
        if j % 2 == 0:
            acc0 = acc0 + p
        else:
            acc1 = acc1 + p
    h_ref[...] += acc0 + acc1

    @pl.when(kg == NKB - 1)
    def _fin():
        h = h_ref[...]
        coef = jnp.dot(h.astype(jnp.bfloat16), b2_ref[...],
                       preferred_element_type=jnp.float32)
        cbf = coef.astype(jnp.bfloat16)
        m2 = []
        for u in range(U):
            pu = jnp.dot(cbf[:, u * K:(u + 1) * K],
                         b2t_ref[u * K:(u + 1) * K, :],
                         preferred_element_type=jnp.float32)
            p2_ref[:, u * D:(u + 1) * D] = pu
            m2.append(jnp.sum(pu * pu, axis=1))
        mags2 = jnp.stack(m2, axis=1)
        step_on = d_const < depth_ref[0]
        avail = avail_ref[...]
        masked = jnp.where(avail > 0.5, mags2, -jnp.inf)
        m = jnp.max(masked, axis=1, keepdims=True)
        iota = lax.broadcasted_iota(jnp.int32, (B, U), 1)
        idx = jnp.min(jnp.where(masked == m, iota, U + 1), axis=1)
        onehot = iota == idx[:, None]
        avail_o_ref[...] = jnp.where(
            jnp.logical_and(step_on, onehot), 0.0, avail)
        idxp_ref[:, 0:U] = jnp.zeros((B, U), jnp.int32) + idx[:, None]
        idx2_ref[...] = idx[:, None]
        route_ref[...] = jnp.where(step_on, idx, 0)[:, None]


def _hash_route_call(d, depth_arr, x3, hw, B2bf, B2Tbf, avail, cast_hw=False):
    full = lambda k: (0, 0)
    out_specs = [
        pl.BlockSpec((B, D), full),
        pl.BlockSpec((B, UD), full),
        pl.BlockSpec((B, 128), full),
        pl.BlockSpec((B, 1), full),
        pl.BlockSpec((B, 1), full),
        pl.BlockSpec((B, U), full),
    ]
    out_shape = [
        jax.ShapeDtypeStruct((B, D), jnp.float32),
        jax.ShapeDtypeStruct((B, UD), jnp.float32),
        jax.ShapeDtypeStruct((B, 128), jnp.int32),
        jax.ShapeDtypeStruct((B, 1), jnp.int32),
        jax.ShapeDtypeStruct((B, 1), jnp.int32),
        jax.ShapeDtypeStruct((B, U), jnp.float32),
    ]
    if cast_hw:
        out_specs.append(pl.BlockSpec((CBLK * HW, D), lambda k: (k, 0)))
        out_shape.append(jax.ShapeDtypeStruct((F, D), jnp.bfloat16))
    return pl.pallas_call(
        functools.partial(_hash_route_kernel, d, cast_hw),
        grid=(NKB,),
        in_specs=[
            pl.BlockSpec(memory_space=pltpu.SMEM),
            pl.BlockSpec((B, CBLK, HW), lambda k: (0, k, 0)),
            pl.BlockSpec((CBLK * HW, D), lambda k: (k, 0)),
            pl.BlockSpec((D, UK), full),
            pl.BlockSpec((UK, D), full),
            pl.BlockSpec((B, U), full),
        ],
        out_specs=out_specs,
        out_shape=out_shape,
    )(depth_arr, x3, hw, B2bf, B2Tbf, avail)


NW = 32
EPW = B // NW


def _sc_accum_call(d_const, depth16, idxp_f, h_f, p2_f, outp_f, act_f):
    mesh = plsc.VectorSubcoreMesh(core_axis_name="core",
                                  subcore_axis_name="subcore")
    cp = pltpu.CompilerParams()
    if "needs_layout_passes" in pltpu.CompilerParams.__dataclass_fields__:
        cp = dataclasses.replace(cp, needs_layout_passes=False)

    @functools.partial(
        pl.kernel, mesh=mesh, compiler_params=cp,
        out_type=[
            jax.ShapeDtypeStruct((B * D,), jnp.float32),
            jax.ShapeDtypeStruct((B * U,), jnp.float32),
        ],
        scratch_types=[
            pltpu.VMEM((U,), jnp.int32),
            pltpu.VMEM((EPW * 128,), jnp.int32),
            pltpu.VMEM((EPW * U,), jnp.float32),
            pltpu.VMEM((EPW * D,), jnp.float32),
            pltpu.VMEM((EPW * D,), jnp.float32),
            pltpu.VMEM((EPW * UD,), jnp.float32),
            pltpu.VMEM((EPW * D,), jnp.float32),
            pltpu.VMEM((EPW * U,), jnp.float32),
            pltpu.SemaphoreType.DMA,
        ],
    )
    def sc_accum(depth_hbm, idxp_hbm, h_hbm, p2_hbm, outp_hbm,
                 act_hbm, outp_o, act_o,
                 d_v, ix_v, ac_v, h_v, o_v, p2_v, on_v, acn_v, sem):
        wid = lax.axis_index("subcore") * 2 + lax.axis_index("core")
        b0 = wid * EPW
        cps = [
            pltpu.async_copy(idxp_hbm.at[pl.ds(b0 * 128, EPW * 128)], ix_v,
                             sem),
            pltpu.async_copy(act_hbm.at[pl.ds(b0 * U, EPW * U)], ac_v, sem),
            pltpu.async_copy(h_hbm.at[pl.ds(b0 * D, EPW * D)], h_v, sem),
            pltpu.async_copy(outp_hbm.at[pl.ds(b0 * D, EPW * D)], o_v, sem),
            pltpu.async_copy(p2_hbm.at[pl.ds(b0 * UD, EPW * UD)], p2_v, sem),
        ]
        pltpu.sync_copy(depth_hbm, d_v)
        for cp in cps:
            cp.wait()
        son = d_const < d_v[...]
        sonf = jnp.where(son, 1.0, 0.0)
        iota = lax.broadcasted_iota(jnp.int32, (U,), 0)
        for e in range(EPW):
            idxv = ix_v[pl.ds(e * 128, U)]
            act = ac_v[pl.ds(e * U, U)]
            actson = act * sonf
            rn2 = jnp.zeros((U,), jnp.float32)
            for j in range(D // U):
                hj = h_v[pl.ds(e * D + U * j, U)]
                pj = plsc.load_gather(
                    p2_v, [e * UD + idxv * D + U * j + iota])
                r = hj - pj
                rn2 = rn2 + r * r
                on_v[pl.ds(e * D + U * j, U)] = (
                    o_v[pl.ds(e * D + U * j, U)] + r * actson)
            rtot = jnp.sum(rn2)
            live = rtot > 1e-10
            acn_v[pl.ds(e * U, U)] = jnp.where(
                son, act * jnp.where(live, 1.0, 0.0), act)
        co = pltpu.async_copy(on_v, outp_o.at[pl.ds(b0 * D, EPW * D)], sem)
        ca = pltpu.async_copy(acn_v, act_o.at[pl.ds(b0 * U, EPW * U)], sem)
        co.wait()
        ca.wait()

    return sc_accum(depth16, idxp_f, h_f, p2_f, outp_f, act_f)


GB = 8


def _conv_kernel(idx_ref, x_ref, w_ref, o_ref):
    g = pl.program_id(0)
    wpos = lax.broadcasted_iota(jnp.int32, (C, HW), 1) % 32
    mnr = (wpos != 31).astype(jnp.bfloat16)
    mnl = (wpos != 0).astype(jnp.bfloat16)
    for e in range(GB):
        ib = idx_ref[g * GB + e]
        xb = x_ref[e].astype(jnp.bfloat16)
        acc = jnp.zeros((C, HW), jnp.float32)
        acc1 = jnp.zeros((C, HW), jnp.float32)
        for t in range(9):
            ky, kx = t // 3, t % 3
            dy, dx = ky - 1, kx - 1
            sh = 32 * dy + dx
            if sh > 0:
                s = jnp.concatenate(
                    [xb[:, sh:], jnp.zeros((C, sh), jnp.bfloat16)], axis=1)
            elif sh < 0:
                s = jnp.concatenate(
                    [jnp.zeros((C, -sh), jnp.bfloat16), xb[:, :sh]], axis=1)
            else:
                s = xb
            if dx == 1:
                s = s * mnr
            elif dx == -1:
                s = s * mnl
            p = jnp.dot(w_ref[ib, t], s,
                        preferred_element_type=jnp.float32)
            if t % 2 == 0:
                acc = acc + p
            else:
                acc1 = acc1 + p
        o_ref[e] = jnp.maximum(acc + acc1, 0.0)


def _conv_call(idx, x3, CWr_bf):
    return pl.pallas_call(
        _conv_kernel,
        grid_spec=pltpu.PrefetchScalarGridSpec(
            num_scalar_prefetch=1,
            grid=(B // GB,),
            in_specs=[
                pl.BlockSpec((GB, C, HW), lambda g, idx_ref: (g, 0, 0)),
                pl.BlockSpec((U, 9, C, C),
                             lambda g, idx_ref: (0, 0, 0, 0)),
            ],
            out_specs=pl.BlockSpec((GB, C, HW), lambda g, idx_ref: (g, 0, 0)),
        ),
        out_shape=jax.ShapeDtypeStruct((B, C, HW), jnp.float32),
    )(idx, x3, CWr_bf)


def kernel(x, hash_W, bases, conv_W, depth):
    depth16 = jnp.full((U,), depth, jnp.int32)
    depth_arr = jnp.asarray(depth, jnp.int32).reshape(1)
    x3 = x.reshape(B, C, HW)
    B2bf = bases.transpose(1, 0, 2).reshape(D, UK).astype(jnp.bfloat16)
    B2Tbf = bases.transpose(0, 2, 1).reshape(UK, D).astype(jnp.bfloat16)
    CWr_bf = conv_W.transpose(0, 3, 4, 1, 2).reshape(U, 9, C, C).astype(
        jnp.bfloat16)

    outp_f = jnp.zeros((B * D,), jnp.float32)
    avail = jnp.ones((B, U), jnp.float32)
    act_f = jnp.ones((B * U,), jnp.float32)
    routes = []
    xcur = x3
    hw_bf = None
    for d in range(DEPTH):
        if d == 0:
            h, p2, idxp, idx2, route_d, avail, hw_bf = _hash_route_call(
                d, depth_arr, xcur, hash_W, B2bf, B2Tbf, avail, cast_hw=True)
        else:
            h, p2, idxp, idx2, route_d, avail = _hash_route_call(
                d, depth_arr, xcur, hw_bf, B2bf, B2Tbf, avail)
        routes.append(route_d[:, 0])
        if d < DEPTH - 1:
            xcur = _conv_call(idx2[:, 0], xcur, CWr_bf)
        outp_f, act_f = _sc_accum_call(
            d, depth16, idxp.reshape(B * 128), h.reshape(B * D),
            p2.reshape(B * UD), outp_f, act_f)
    return outp_f.reshape(B, D), jnp.stack(routes, axis=1)


# speedup vs baseline: 1.4081x; 1.1949x over previous
import functools

import jax
import jax.numpy as jnp
from jax import lax
from jax.experimental import pallas as pl
from jax.experimental.pallas import tpu as pltpu

B = 64
U = 16
D = 256
K = 32
C = 32
HW = 1024
F = C * HW
DEPTH = 4
UK = U * K
UD = U * D
CBLK = 8
NKB = C // CBLK


def _hash_route_kernel(d_const, cast_hw, *refs):
    if cast_hw:
        (depth_ref, x_ref, hw_ref, b2_ref, b2t_ref,
         avail_ref, outp_ref, act_ref,
         h_ref, route_ref, idx_ref, avail_o_ref, out_o_ref, act_o_ref,
         hwbf_ref) = refs
    else:
        (depth_ref, x_ref, hw_ref, b2_ref, b2t_ref,
         avail_ref, outp_ref, act_ref,
         h_ref, route_ref, idx_ref, avail_o_ref, out_o_ref,
         act_o_ref) = refs
    kg = pl.program_id(0)

    @pl.when(kg == 0)
    def _init():
        h_ref[...] = jnp.zeros_like(h_ref)

    if cast_hw:
        hwc = hw_ref[...].astype(jnp.bfloat16)
        hwbf_ref[...] = hwc
    else:
        hwc = hw_ref[...]
    acc = jnp.zeros((B, D), jnp.float32)
    for j in range(CBLK):
        xc = x_ref[:, j, :].astype(jnp.bfloat16)
        acc = acc + jnp.dot(xc, hwc[j * HW:(j + 1) * HW, :],
                            preferred_element_type=jnp.float32)
    h_ref[...] += acc

    @pl.when(kg == NKB - 1)
    def _fin():
        h = h_ref[...]
        coef = jnp.dot(h.astype(jnp.bfloat16), b2_ref[...],
                       preferred_element_type=jnp.float32)
        cbf = coef.astype(jnp.bfloat16)
        projs = []
        m2 = []
        for u in range(U):
            pu = jnp.dot(cbf[:, u * K:(u + 1) * K],
                         b2t_ref[u * K:(u + 1) * K, :],
                         preferred_element_type=jnp.float32)
            projs.append(pu)
            m2.append(jnp.sum(pu * pu, axis=1))
        mags2 = jnp.stack(m2, axis=1)
        step_on = d_const < depth_ref[0]
        avail = avail_ref[...]
        masked = jnp.where(avail > 0.5, mags2, -jnp.inf)
        m = jnp.max(masked, axis=1, keepdims=True)
        iota = lax.broadcasted_iota(jnp.int32, (B, U), 1)
        idx = jnp.min(jnp.where(masked == m, iota, U + 1), axis=1)
        onehot = (iota == idx[:, None]).astype(jnp.float32)
        proj = jnp.zeros((B, D), jnp.float32)
        for u in range(U):
            proj = proj + projs[u] * onehot[:, u][:, None]
        residual = h - proj
        act = act_ref[...][:, 0]
        sonf = jnp.where(step_on, 1.0, 0.0)
        out_o_ref[...] = outp_ref[...] + residual * (act * sonf)[:, None]
        rnorm2 = jnp.sum(residual * residual, axis=1)
        live = (rnorm2 > 1e-10).astype(jnp.float32)
        act_o_ref[...] = jnp.where(step_on, act * live, act)[:, None]
        avail_o_ref[...] = jnp.where(step_on, avail * (1.0 - onehot), avail)
        route_ref[...] = jnp.where(step_on, idx, 0)[:, None]
        idx_ref[...] = idx[:, None]


def _hash_route_call(d, depth_arr, x3, hw, B2bf, B2Tbf, avail, outp, act,
                     cast_hw=False):
    full = lambda k: (0, 0)
    out_specs = [
        pl.BlockSpec((B, D), full),
        pl.BlockSpec((B, 1), full),
        pl.BlockSpec((B, 1), full),
        pl.BlockSpec((B, U), full),
        pl.BlockSpec((B, D), full),
        pl.BlockSpec((B, 1), full),
    ]
    out_shape = [
        jax.ShapeDtypeStruct((B, D), jnp.float32),
        jax.ShapeDtypeStruct((B, 1), jnp.int32),
        jax.ShapeDtypeStruct((B, 1), jnp.int32),
        jax.ShapeDtypeStruct((B, U), jnp.float32),
        jax.ShapeDtypeStruct((B, D), jnp.float32),
        jax.ShapeDtypeStruct((B, 1), jnp.float32),
    ]
    if cast_hw:
        out_specs.append(pl.BlockSpec((CBLK * HW, D), lambda k: (k, 0)))
        out_shape.append(jax.ShapeDtypeStruct((F, D), jnp.bfloat16))
    return pl.pallas_call(
        functools.partial(_hash_route_kernel, d, cast_hw),
        grid=(NKB,),
        in_specs=[
            pl.BlockSpec(memory_space=pltpu.SMEM),
            pl.BlockSpec((B, CBLK, HW), lambda k: (0, k, 0)),
            pl.BlockSpec((CBLK * HW, D), lambda k: (k, 0)),
            pl.BlockSpec((D, UK), full),
            pl.BlockSpec((UK, D), full),
            pl.BlockSpec((B, U), full),
            pl.BlockSpec((B, D), full),
            pl.BlockSpec((B, 1), full),
        ],
        out_specs=out_specs,
        out_shape=out_shape,
    )(depth_arr, x3, hw, B2bf, B2Tbf, avail, outp, act)


GB = 8


def _conv_kernel(idx_ref, x_ref, w_ref, o_ref):
    g = pl.program_id(0)
    wpos = lax.broadcasted_iota(jnp.int32, (C, HW), 1) % 32
    mnr = (wpos != 31).astype(jnp.bfloat16)
    mnl = (wpos != 0).astype(jnp.bfloat16)
    for e in range(GB):
        ib = idx_ref[g * GB + e]
        xb = x_ref[e].astype(jnp.bfloat16)
        taps = []
        for t in range(9):
            ky, kx = t // 3, t % 3
            dy, dx = ky - 1, kx - 1
            sh = 32 * dy + dx
            if sh > 0:
                s = jnp.concatenate(
                    [xb[:, sh:], jnp.zeros((C, sh), jnp.bfloat16)], axis=1)
            elif sh < 0:
                s = jnp.concatenate(
                    [jnp.zeros((C, -sh), jnp.bfloat16), xb[:, :sh]], axis=1)
            else:
                s = xb
            if dx == 1:
                s = s * mnr
            elif dx == -1:
                s = s * mnl
            taps.append(s)
        patch = jnp.concatenate(taps, axis=0)
        acc = jnp.dot(w_ref[ib], patch,
                      preferred_element_type=jnp.float32)
        o_ref[e] = jnp.maximum(acc, 0.0)


def _conv_call(idx, x3, CWr_bf):
    return pl.pallas_call(
        _conv_kernel,
        grid_spec=pltpu.PrefetchScalarGridSpec(
            num_scalar_prefetch=1,
            grid=(B // GB,),
            in_specs=[
                pl.BlockSpec((GB, C, HW), lambda g, idx_ref: (g, 0, 0)),
                pl.BlockSpec((U, C, 9 * C),
                             lambda g, idx_ref: (0, 0, 0)),
            ],
            out_specs=pl.BlockSpec((GB, C, HW), lambda g, idx_ref: (g, 0, 0)),
        ),
        out_shape=jax.ShapeDtypeStruct((B, C, HW), jnp.float32),
    )(idx, x3, CWr_bf)


def kernel(x, hash_W, bases, conv_W, depth):
    depth_arr = jnp.asarray(depth, jnp.int32).reshape(1)
    x3 = x.reshape(B, C, HW)
    B2bf = bases.transpose(1, 0, 2).reshape(D, UK).astype(jnp.bfloat16)
    B2Tbf = bases.transpose(0, 2, 1).reshape(UK, D).astype(jnp.bfloat16)
    CWr_bf = conv_W.transpose(0, 1, 3, 4, 2).reshape(U, C, 9 * C).astype(
        jnp.bfloat16)

    outp = jnp.zeros((B, D), jnp.float32)
    avail = jnp.ones((B, U), jnp.float32)
    act = jnp.ones((B, 1), jnp.float32)
    routes = []
    xcur = x3
    hw_bf = None
    for d in range(DEPTH):
        if d == 0:
            (h, route_d, idx2, avail, outp, act,
             hw_bf) = _hash_route_call(
                d, depth_arr, xcur, hash_W, B2bf, B2Tbf, avail, outp, act,
                cast_hw=True)
        else:
            h, route_d, idx2, avail, outp, act = _hash_route_call(
                d, depth_arr, xcur, hw_bf, B2bf, B2Tbf, avail, outp, act)
        routes.append(route_d[:, 0])
        if d < DEPTH - 1:
            xcur = _conv_call(idx2[:, 0], xcur, CWr_bf)
    return outp, jnp.stack(routes, axis=1)
